# baseline (speedup 1.0000x reference)
import functools

import jax
import jax.numpy as jnp
from jax.experimental import pallas as pl
from jax.experimental.pallas import tpu as pltpu


def _make_body(n, d_out, m_blk, depth):
    n_blk = n // m_blk

    def _body(adj_hbm, x_hbm, w_ref, out_hbm,
              xbuf, buf, obuf, sems, osems, xsem):
        def cp_adj(i, slot):
            return pltpu.make_async_copy(
                adj_hbm.at[pl.ds(i * m_blk, m_blk), :],
                buf.at[slot],
                sems.at[slot],
            )

        def cp_out(i, oslot):
            return pltpu.make_async_copy(
                obuf.at[oslot],
                out_hbm.at[pl.ds(i * m_blk, m_blk), :],
                osems.at[oslot],
            )

        cp_x = pltpu.make_async_copy(x_hbm, xbuf, xsem)
        cp_x.start()

        for s in range(depth - 1):
            cp_adj(s, s).start()

        def step(i, carry):
            nxt = i + depth - 1

            @pl.when(nxt < n_blk)
            def _start_next():
                cp_adj(nxt, jax.lax.rem(nxt, depth)).start()

            slot = jax.lax.rem(i, depth)
            cp_adj(i, slot).wait()

            @pl.when(i == 0)
            def _wait_x():
                cp_x.wait()

            h = jnp.dot(buf[slot], xbuf[...],
                        preferred_element_type=jnp.float32)

            oslot = jax.lax.rem(i, 2)

            @pl.when(i >= 2)
            def _free_oslot():
                cp_out(i - 2, oslot).wait()

            obuf[oslot] = jax.lax.dot_general(
                h, w_ref[...], (((1,), (1,)), ((), ())),
                preferred_element_type=jnp.float32,
            )
            cp_out(i, oslot).start()
            return carry

        jax.lax.fori_loop(0, n_blk, step, 0)

        for tail in (n_blk - 2, n_blk - 1):
            if tail >= 0:
                cp_out(tail, tail % 2).wait()

    return _body


@functools.partial(jax.jit, static_argnames=("m_blk", "depth", "interpret"))
def _graph_conv(x, adj, W, *, m_blk, depth, interpret=False):
    n, d_in = x.shape
    d_out = W.shape[0]
    return pl.pallas_call(
        _make_body(n, d_out, m_blk, depth),
        in_specs=[
            pl.BlockSpec(memory_space=pltpu.MemorySpace.HBM),
            pl.BlockSpec(memory_space=pltpu.MemorySpace.HBM),
            pl.BlockSpec(memory_space=pltpu.MemorySpace.VMEM),
        ],
        out_specs=pl.BlockSpec(memory_space=pltpu.MemorySpace.HBM),
        out_shape=jax.ShapeDtypeStruct((n, d_out), jnp.float32),
        scratch_shapes=[
            pltpu.VMEM((n, d_in), jnp.float32),
            pltpu.VMEM((depth, m_blk, n), jnp.float32),
            pltpu.VMEM((2, m_blk, d_out), jnp.float32),
            pltpu.SemaphoreType.DMA((depth,)),
            pltpu.SemaphoreType.DMA((2,)),
            pltpu.SemaphoreType.DMA,
        ],
        compiler_params=pltpu.CompilerParams(
            vmem_limit_bytes=64 * 1024 * 1024),
        interpret=interpret,
    )(adj, x, W)


def kernel(x, adj, W):
    n = x.shape[0]
    m_blk = 200 if n % 200 == 0 else n
    depth = 6 if n // m_blk >= 6 else 1
    return _graph_conv(x, adj, W, m_blk=m_blk, depth=depth)
